# baseline (speedup 1.0000x reference)
import functools

import jax
import jax.numpy as jnp
from jax import lax
from jax.experimental import pallas as pl
from jax.experimental.pallas import tpu as pltpu
from jax.experimental.pallas import tpu_sc as plsc

NLABELS = 100000
EMBED_DIM = 128
BATCH = 4096

_L = 16
_NW = 32
_BPW = BATCH // _NW
_CHUNKS = EMBED_DIM // _L
_NCH = 8
_RPC = _BPW // _NCH
_UNROLL = 4

_GDN = lax.GatherDimensionNumbers(
    offset_dims=(), collapsed_slice_dims=(0,), start_index_map=(0,)
)


def _permute(v, idx):
    return lax.gather(
        v,
        idx[:, None],
        dimension_numbers=_GDN,
        slice_sizes=(1,),
        mode=lax.GatherScatterMode.PROMISE_IN_BOUNDS,
    )


def _lane_sum(v):
    lanes = lax.iota(jnp.int32, _L)
    for sh in (8, 4, 2, 1):
        v = v + _permute(v, lanes ^ sh)
    return v


def _rsqrt(s):
    i = plsc.bitcast(s, jnp.int32)
    i = jnp.int32(0x5F3759DF) - (i >> 1)
    y = plsc.bitcast(i, jnp.float32)
    return y * (1.5 - (s * 0.5) * y * y)


def _normalize_row(rows_v, r):
    chunks = [rows_v[r, pl.ds(c * _L, _L)] for c in range(_CHUNKS)]
    acc = chunks[0] * chunks[0]
    for c in range(1, _CHUNKS):
        acc = acc + chunks[c] * chunks[c]
    scale = _rsqrt(_lane_sum(acc))
    for c in range(_CHUNKS):
        rows_v[r, pl.ds(c * _L, _L)] = chunks[c] * scale


def _body(y_hbm, table_hbm, out_hbm, idx_v, rows_v, gsems, osems):
    wid = lax.axis_index("s") * 2 + lax.axis_index("c")
    base = wid * _BPW
    pltpu.sync_copy(y_hbm.at[pl.ds(base, _BPW)], idx_v)
    gathers = [
        pltpu.async_copy(
            table_hbm.at[idx_v.at[pl.ds(ch * _RPC, _RPC)]],
            rows_v.at[pl.ds(ch * _RPC, _RPC)],
            gsems.at[ch],
        )
        for ch in range(_NCH)
    ]
    writes = []
    for ch in range(_NCH):
        gathers[ch].wait()

        def group(i, _, ch=ch):
            for k in range(_UNROLL):
                _normalize_row(rows_v, ch * _RPC + i * _UNROLL + k)
            return _

        lax.fori_loop(0, _RPC // _UNROLL, group, None)
        writes.append(
            pltpu.async_copy(
                rows_v.at[pl.ds(ch * _RPC, _RPC)],
                out_hbm.at[pl.ds(base + ch * _RPC, _RPC)],
                osems.at[ch],
            )
        )
    for w in writes:
        w.wait()


@jax.jit
def kernel(y, table):
    mesh = plsc.VectorSubcoreMesh(core_axis_name="c", subcore_axis_name="s")
    f = functools.partial(
        pl.kernel,
        mesh=mesh,
        out_type=jax.ShapeDtypeStruct((BATCH, EMBED_DIM), jnp.float32),
        scratch_types=[
            pltpu.VMEM((_BPW,), jnp.int32),
            pltpu.VMEM((_BPW, EMBED_DIM), jnp.float32),
            pltpu.SemaphoreType.DMA((_NCH,)),
            pltpu.SemaphoreType.DMA((_NCH,)),
        ],
        compiler_params=pltpu.CompilerParams(needs_layout_passes=False),
    )(_body)
    return f(y.astype(jnp.int32), table)


# speedup vs baseline: 1.0467x; 1.0467x over previous
import functools

import jax
import jax.numpy as jnp
from jax import lax
from jax.experimental import pallas as pl
from jax.experimental.pallas import tpu as pltpu
from jax.experimental.pallas import tpu_sc as plsc

NLABELS = 100000
EMBED_DIM = 128
BATCH = 4096

_L = 16
_NW = 32
_BPW = BATCH // _NW
_CHUNKS = EMBED_DIM // _L
_NCH = 4
_RPC = _BPW // _NCH
_UNROLL = 4

_GDN = lax.GatherDimensionNumbers(
    offset_dims=(), collapsed_slice_dims=(0,), start_index_map=(0,)
)


def _permute(v, idx):
    return lax.gather(
        v,
        idx[:, None],
        dimension_numbers=_GDN,
        slice_sizes=(1,),
        mode=lax.GatherScatterMode.PROMISE_IN_BOUNDS,
    )


def _lane_sum(v):
    lanes = lax.iota(jnp.int32, _L)
    for sh in (8, 4, 2, 1):
        v = v + _permute(v, lanes ^ sh)
    return v


def _rsqrt(s):
    i = plsc.bitcast(s, jnp.int32)
    i = jnp.int32(0x5F3759DF) - (i >> 1)
    y = plsc.bitcast(i, jnp.float32)
    return y * (1.5 - (s * 0.5) * y * y)


def _normalize_row(rows_v, r):
    chunks = [rows_v[r, pl.ds(c * _L, _L)] for c in range(_CHUNKS)]
    acc = chunks[0] * chunks[0]
    for c in range(1, _CHUNKS):
        acc = acc + chunks[c] * chunks[c]
    scale = _rsqrt(_lane_sum(acc))
    for c in range(_CHUNKS):
        rows_v[r, pl.ds(c * _L, _L)] = chunks[c] * scale


def _body(y_hbm, table_hbm, out_hbm, idx_v, rows_v, gsems, osems):
    wid = lax.axis_index("s") * 2 + lax.axis_index("c")
    base = wid * _BPW
    pltpu.sync_copy(y_hbm.at[pl.ds(base, _BPW)], idx_v)
    gathers = [
        pltpu.async_copy(
            table_hbm.at[idx_v.at[pl.ds(ch * _RPC, _RPC)]],
            rows_v.at[pl.ds(ch * _RPC, _RPC)],
            gsems.at[ch],
        )
        for ch in range(_NCH)
    ]
    writes = []
    for ch in range(_NCH):
        gathers[ch].wait()

        def group(i, _, ch=ch):
            for k in range(_UNROLL):
                _normalize_row(rows_v, ch * _RPC + i * _UNROLL + k)
            return _

        lax.fori_loop(0, _RPC // _UNROLL, group, None)
        writes.append(
            pltpu.async_copy(
                rows_v.at[pl.ds(ch * _RPC, _RPC)],
                out_hbm.at[pl.ds(base + ch * _RPC, _RPC)],
                osems.at[ch],
            )
        )
    for w in writes:
        w.wait()


@jax.jit
def kernel(y, table):
    mesh = plsc.VectorSubcoreMesh(core_axis_name="c", subcore_axis_name="s")
    f = functools.partial(
        pl.kernel,
        mesh=mesh,
        out_type=jax.ShapeDtypeStruct((BATCH, EMBED_DIM), jnp.float32),
        scratch_types=[
            pltpu.VMEM((_BPW,), jnp.int32),
            pltpu.VMEM((_BPW, EMBED_DIM), jnp.float32),
            pltpu.SemaphoreType.DMA((_NCH,)),
            pltpu.SemaphoreType.DMA((_NCH,)),
        ],
        compiler_params=pltpu.CompilerParams(needs_layout_passes=False),
    )(_body)
    return f(y.astype(jnp.int32), table)


# speedup vs baseline: 1.0502x; 1.0033x over previous
import functools

import jax
import jax.numpy as jnp
from jax import lax
from jax.experimental import pallas as pl
from jax.experimental.pallas import tpu as pltpu
from jax.experimental.pallas import tpu_sc as plsc

NLABELS = 100000
EMBED_DIM = 128
BATCH = 4096

_L = 16
_NW = 32
_BPW = BATCH // _NW
_CHUNKS = EMBED_DIM // _L
_CHUNK_ROWS = (48, 40, 24, 16)
_CHUNK_OFF = (0, 48, 88, 112)
_NCH = len(_CHUNK_ROWS)
_UNROLL = 4

_GDN = lax.GatherDimensionNumbers(
    offset_dims=(), collapsed_slice_dims=(0,), start_index_map=(0,)
)


def _permute(v, idx):
    return lax.gather(
        v,
        idx[:, None],
        dimension_numbers=_GDN,
        slice_sizes=(1,),
        mode=lax.GatherScatterMode.PROMISE_IN_BOUNDS,
    )


def _lane_sum(v):
    lanes = lax.iota(jnp.int32, _L)
    for sh in (8, 4, 2, 1):
        v = v + _permute(v, lanes ^ sh)
    return v


def _rsqrt(s):
    i = plsc.bitcast(s, jnp.int32)
    i = jnp.int32(0x5F3759DF) - (i >> 1)
    y = plsc.bitcast(i, jnp.float32)
    return y * (1.5 - (s * 0.5) * y * y)


def _normalize_row(rows_v, r):
    chunks = [rows_v[r, pl.ds(c * _L, _L)] for c in range(_CHUNKS)]
    acc = chunks[0] * chunks[0]
    for c in range(1, _CHUNKS):
        acc = acc + chunks[c] * chunks[c]
    scale = _rsqrt(_lane_sum(acc))
    for c in range(_CHUNKS):
        rows_v[r, pl.ds(c * _L, _L)] = chunks[c] * scale


def _body(y_hbm, table_hbm, out_hbm, idx_v, rows_v, gsems, osems):
    wid = lax.axis_index("s") * 2 + lax.axis_index("c")
    base = wid * _BPW
    pltpu.sync_copy(y_hbm.at[pl.ds(base, _BPW)], idx_v)
    gathers = [
        pltpu.async_copy(
            table_hbm.at[idx_v.at[pl.ds(_CHUNK_OFF[ch], _CHUNK_ROWS[ch])]],
            rows_v.at[pl.ds(_CHUNK_OFF[ch], _CHUNK_ROWS[ch])],
            gsems.at[ch],
        )
        for ch in range(_NCH)
    ]
    writes = []
    for ch in range(_NCH):
        gathers[ch].wait()
        off = _CHUNK_OFF[ch]

        def group(i, _, off=off):
            for k in range(_UNROLL):
                _normalize_row(rows_v, off + i * _UNROLL + k)
            return _

        lax.fori_loop(0, _CHUNK_ROWS[ch] // _UNROLL, group, None)
        writes.append(
            pltpu.async_copy(
                rows_v.at[pl.ds(off, _CHUNK_ROWS[ch])],
                out_hbm.at[pl.ds(base + off, _CHUNK_ROWS[ch])],
                osems.at[ch],
            )
        )
    for w in writes:
        w.wait()


@jax.jit
def kernel(y, table):
    mesh = plsc.VectorSubcoreMesh(core_axis_name="c", subcore_axis_name="s")
    f = functools.partial(
        pl.kernel,
        mesh=mesh,
        out_type=jax.ShapeDtypeStruct((BATCH, EMBED_DIM), jnp.float32),
        scratch_types=[
            pltpu.VMEM((_BPW,), jnp.int32),
            pltpu.VMEM((_BPW, EMBED_DIM), jnp.float32),
            pltpu.SemaphoreType.DMA((_NCH,)),
            pltpu.SemaphoreType.DMA((_NCH,)),
        ],
        compiler_params=pltpu.CompilerParams(needs_layout_passes=False),
    )(_body)
    return f(y.astype(jnp.int32), table)


# speedup vs baseline: 1.0675x; 1.0165x over previous
import functools

import jax
import jax.numpy as jnp
from jax import lax
from jax.experimental import pallas as pl
from jax.experimental.pallas import tpu as pltpu
from jax.experimental.pallas import tpu_sc as plsc

NLABELS = 100000
EMBED_DIM = 128
BATCH = 4096

_L = 16
_NW = 32
_BPW = BATCH // _NW
_CHUNKS = EMBED_DIM // _L
_CHUNK_ROWS = (56, 40, 24, 8)
_CHUNK_OFF = (0, 56, 96, 120)
_NCH = len(_CHUNK_ROWS)
_UNROLL = 4

_GDN = lax.GatherDimensionNumbers(
    offset_dims=(), collapsed_slice_dims=(0,), start_index_map=(0,)
)


def _permute(v, idx):
    return lax.gather(
        v,
        idx[:, None],
        dimension_numbers=_GDN,
        slice_sizes=(1,),
        mode=lax.GatherScatterMode.PROMISE_IN_BOUNDS,
    )


def _lane_sum(v):
    lanes = lax.iota(jnp.int32, _L)
    for sh in (8, 4, 2, 1):
        v = v + _permute(v, lanes ^ sh)
    return v


def _rsqrt(s):
    i = plsc.bitcast(s, jnp.int32)
    i = jnp.int32(0x5F3759DF) - (i >> 1)
    y = plsc.bitcast(i, jnp.float32)
    return y * (1.5 - (s * 0.5) * y * y)


def _normalize_row(rows_v, r):
    chunks = [rows_v[r, pl.ds(c * _L, _L)] for c in range(_CHUNKS)]
    acc = chunks[0] * chunks[0]
    for c in range(1, _CHUNKS):
        acc = acc + chunks[c] * chunks[c]
    scale = _rsqrt(_lane_sum(acc))
    for c in range(_CHUNKS):
        rows_v[r, pl.ds(c * _L, _L)] = chunks[c] * scale


def _body(y_hbm, table_hbm, out_hbm, idx_v, rows_v, gsems, osems):
    wid = lax.axis_index("s") * 2 + lax.axis_index("c")
    base = wid * _BPW
    n0 = _CHUNK_ROWS[0]
    pltpu.sync_copy(y_hbm.at[pl.ds(base, n0)], idx_v.at[pl.ds(0, n0)])
    gathers = [
        pltpu.async_copy(
            table_hbm.at[idx_v.at[pl.ds(0, n0)]],
            rows_v.at[pl.ds(0, n0)],
            gsems.at[0],
        )
    ]
    pltpu.sync_copy(
        y_hbm.at[pl.ds(base + n0, _BPW - n0)], idx_v.at[pl.ds(n0, _BPW - n0)]
    )
    gathers += [
        pltpu.async_copy(
            table_hbm.at[idx_v.at[pl.ds(_CHUNK_OFF[ch], _CHUNK_ROWS[ch])]],
            rows_v.at[pl.ds(_CHUNK_OFF[ch], _CHUNK_ROWS[ch])],
            gsems.at[ch],
        )
        for ch in range(1, _NCH)
    ]
    writes = []
    for ch in range(_NCH):
        gathers[ch].wait()
        off = _CHUNK_OFF[ch]

        def group(i, _, off=off):
            for k in range(_UNROLL):
                _normalize_row(rows_v, off + i * _UNROLL + k)
            return _

        lax.fori_loop(0, _CHUNK_ROWS[ch] // _UNROLL, group, None)
        writes.append(
            pltpu.async_copy(
                rows_v.at[pl.ds(off, _CHUNK_ROWS[ch])],
                out_hbm.at[pl.ds(base + off, _CHUNK_ROWS[ch])],
                osems.at[ch],
            )
        )
    for w in writes:
        w.wait()


@jax.jit
def kernel(y, table):
    mesh = plsc.VectorSubcoreMesh(core_axis_name="c", subcore_axis_name="s")
    f = functools.partial(
        pl.kernel,
        mesh=mesh,
        out_type=jax.ShapeDtypeStruct((BATCH, EMBED_DIM), jnp.float32),
        scratch_types=[
            pltpu.VMEM((_BPW,), jnp.int32),
            pltpu.VMEM((_BPW, EMBED_DIM), jnp.float32),
            pltpu.SemaphoreType.DMA((_NCH,)),
            pltpu.SemaphoreType.DMA((_NCH,)),
        ],
        compiler_params=pltpu.CompilerParams(needs_layout_passes=False),
    )(_body)
    return f(y.astype(jnp.int32), table)
